# Initial kernel scaffold; baseline (speedup 1.0000x reference)
import jax
import jax.numpy as jnp
from jax.experimental import pallas as pl


def kernel(x, rotations, W_match, b_match, W_assembly, b_assembly, W_conv, b_conv):
    raise NotImplementedError("write your pallas kernel here")


# speedup vs baseline: 2.0405x; 2.0405x over previous
import functools
import jax
import jax.numpy as jnp
from jax import lax
from jax.experimental import pallas as pl
from jax.experimental.pallas import tpu as pltpu
from jax.experimental.pallas import tpu_sc as plsc

N_HASHES = 4
CHUNK = 144
L = 6 * 48 * 48
NB = L // CHUNK
HB = 96
PADF = 20128
RNG = 15000
RB = 1000
_INTERPRET = False


def _pc(*a, **k):
    return pl.pallas_call(*a, interpret=_INTERPRET, **k)


def _shifted_slice(ref, s0, i):
    al = s0 & ~7
    r = s0 - al
    v = ref[pl.ds(al + i * RB, RB + 8), :]
    return v[r:r + RB]


def _a1_body(xp_ref, wm_ref, bm_ref, wa_ref, ba_ref, emb_ref):
    i = pl.program_id(0)
    acc = jnp.zeros((RB, 48), jnp.float32)
    for k in range(27):
        dt, r = divmod(k, 9)
        dh, dw = divmod(r, 3)
        off = (dt - 1) * 2500 + (dh - 1) * 50 + (dw - 1)
        sl = _shifted_slice(xp_ref, 64 + 2500 + off, i)
        acc += jnp.dot(sl, wm_ref[k], preferred_element_type=jnp.float32)
    xe = acc + bm_ref[...]
    c = _shifted_slice(xp_ref, 64 + 2500, i)
    ye = jnp.dot(c, wa_ref[...],
                 preferred_element_type=jnp.float32) + ba_ref[...]
    emb_ref[...] = jnp.concatenate(
        [xe, ye, jnp.zeros((RB, 16), jnp.float32)], axis=-1)


def _run_a1(xp, wm, bm, wa, ba):
    return _pc(
        _a1_body,
        grid=(RNG // RB,),
        in_specs=[
            pl.BlockSpec((PADF, 192), lambda i: (0, 0)),
            pl.BlockSpec((27, 192, 48), lambda i: (0, 0, 0)),
            pl.BlockSpec((1, 48), lambda i: (0, 0)),
            pl.BlockSpec((192, 192), lambda i: (0, 0)),
            pl.BlockSpec((1, 192), lambda i: (0, 0)),
        ],
        out_specs=pl.BlockSpec((RB, 256), lambda i: (i, 0)),
        out_shape=jax.ShapeDtypeStruct((RNG, 256), jnp.float32),
    )(xp, wm, bm, wa, ba)


SB = 512


def _a2_body(emb_ref, rot_ref, rl_ref, rg_ref):
    h = pl.program_id(0)
    xe = emb_ref[:, 0:48]
    v = jnp.dot(xe, rot_ref[0], preferred_element_type=jnp.float32)
    vv = jnp.concatenate([v, -v], axis=-1)
    m = jnp.max(vv, axis=-1, keepdims=True)
    lane = lax.broadcasted_iota(jnp.int32, (L, HB), 1)
    code = jnp.min(jnp.where(vv >= m, lane, HB), axis=-1, keepdims=True)
    lane128 = lax.broadcasted_iota(jnp.int32, (L, 128), 1)
    onehot = (code == lane128).astype(jnp.float32)

    r = lax.broadcasted_iota(jnp.int32, (SB, SB), 0)
    c = lax.broadcasted_iota(jnp.int32, (SB, SB), 1)
    ltri = (r >= c).astype(jnp.bfloat16)
    r2 = lax.broadcasted_iota(jnp.int32, (128, 128), 0)
    c2 = lax.broadcasted_iota(jnp.int32, (128, 128), 1)
    sup = (r2 < c2).astype(jnp.float32)

    hist = jnp.sum(onehot, axis=0, keepdims=True)
    less = jnp.dot(hist, sup, preferred_element_type=jnp.float32)
    run = jnp.zeros((1, 128), jnp.float32)
    for b in range(L // SB):
        ob = onehot[b * SB:(b + 1) * SB, :]
        incl = jnp.dot(ltri, ob.astype(jnp.bfloat16),
                       preferred_element_type=jnp.float32)
        val = incl - 1.0 + run + less
        rank = jnp.sum(val * ob, axis=-1, keepdims=True)
        ri = rank.astype(jnp.int32)
        rl_ref[0, pl.ds(b * SB, SB), :] = ri
        rg_ref[0, pl.ds(b * SB, SB), :] = ri + h * L
        run = run + incl[SB - 1:SB, :]


def _run_a2(emb, rot):
    return _pc(
        _a2_body,
        grid=(N_HASHES,),
        in_specs=[
            pl.BlockSpec((L, 256), lambda h: (0, 0)),
            pl.BlockSpec((1, 48, 48), lambda h: (h, 0, 0)),
        ],
        out_specs=[
            pl.BlockSpec((1, L, 1), lambda h: (h, 0, 0)),
            pl.BlockSpec((1, L, 1), lambda h: (h, 0, 0)),
        ],
        out_shape=[
            jax.ShapeDtypeStruct((N_HASHES, L, 1), jnp.int32),
            jax.ShapeDtypeStruct((N_HASHES, L, 1), jnp.int32),
        ],
    )(emb, rot)


def _c_body(e0_ref, em_ref, ep_ref, ret_ref):
    blks = [e0_ref[0, 0], em_ref[0, 0], ep_ref[0, 0]]
    xa = blks[0][:, 0:48]
    xn, ys = [], []
    for t in blks:
        xx = t[:, 0:48]
        nrm = jnp.sqrt(jnp.sum(xx * xx, axis=-1, keepdims=True))
        xn.append(xx / jnp.clip(nrm, 5e-5, None))
        ys.append(t[:, 48:240])
    xmatch = jnp.concatenate(xn, axis=0)
    raw = lax.dot_general(xa, xmatch, (((1,), (1,)), ((), ())),
                          preferred_element_type=jnp.float32)
    m = jnp.max(raw, axis=-1, keepdims=True)
    e = jnp.exp(raw - m)
    s = jnp.sum(e, axis=-1, keepdims=True)
    bs = jnp.log(s) + m
    score = e / s
    ym = jnp.concatenate(ys, axis=0)
    ret = jnp.dot(score, ym, preferred_element_type=jnp.float32)
    ret_ref[0, 0] = jnp.concatenate(
        [ret, bs, jnp.zeros((CHUNK, 63), jnp.float32)], axis=-1)


def _run_c(es4):
    bs_e = lambda f: pl.BlockSpec((1, 1, CHUNK, 256),
                                  lambda h, k: (h, f(k), 0, 0))
    return _pc(
        _c_body,
        grid=(N_HASHES, NB),
        in_specs=[bs_e(lambda k: k),
                  bs_e(lambda k: (k + NB - 1) % NB),
                  bs_e(lambda k: (k + 1) % NB)],
        out_specs=pl.BlockSpec((1, 1, CHUNK, 256), lambda h, k: (h, k, 0, 0)),
        out_shape=jax.ShapeDtypeStruct((N_HASHES, NB, CHUNK, 256),
                                       jnp.float32),
    )(es4, es4, es4)


NW = 32
PB = L // NW
CK = 72
NCK = PB // CK


def _sc_wid():
    return lax.axis_index("s") * 2 + lax.axis_index("c")


def _b_body(emb_hbm, rk_hbm, es_hbm, idx_v, er_v, sem):
    wid = _sc_wid()
    base = wid * PB
    pltpu.sync_copy(emb_hbm.at[pl.ds(base, PB)], er_v)
    for h in range(N_HASHES):
        pltpu.sync_copy(rk_hbm.at[h, wid], idx_v)
        cps = []
        for c in range(NCK):
            cps.append(pltpu.async_copy(
                er_v.at[pl.ds(c * CK, CK)], es_hbm.at[idx_v.at[c]], sem))
        for cp in cps:
            cp.wait()


def _run_b(emb, rk4):
    mesh = plsc.VectorSubcoreMesh(core_axis_name="c", subcore_axis_name="s")
    f = pl.kernel(
        _b_body, mesh=mesh,
        out_type=jax.ShapeDtypeStruct((N_HASHES * L, 256), jnp.float32),
        scratch_types=[pltpu.VMEM((NCK, CK), jnp.int32),
                       pltpu.VMEM((PB, 256), jnp.float32),
                       pltpu.SemaphoreType.DMA],
    )
    return f(emb, rk4)


def _d_body(ret_hbm, rkg_hbm, reto_hbm, idx_v, rows_v, sem):
    wid = _sc_wid()
    base = wid * PB
    for h in range(N_HASHES):
        pltpu.sync_copy(rkg_hbm.at[h, wid], idx_v)
        cps = []
        for c in range(NCK):
            cps.append(pltpu.async_copy(
                ret_hbm.at[idx_v.at[c]], rows_v.at[pl.ds(c * CK, CK)], sem))
        for cp in cps:
            cp.wait()
        pltpu.sync_copy(rows_v, reto_hbm.at[h, pl.ds(base, PB)])


def _run_d(ret_f, rkg4):
    mesh = plsc.VectorSubcoreMesh(core_axis_name="c", subcore_axis_name="s")
    f = pl.kernel(
        _d_body, mesh=mesh,
        out_type=jax.ShapeDtypeStruct((N_HASHES, L, 256), jnp.float32),
        scratch_types=[pltpu.VMEM((NCK, CK), jnp.int32),
                       pltpu.VMEM((PB, 256), jnp.float32),
                       pltpu.SemaphoreType.DMA],
    )
    return f(ret_f, rkg4)


EB = 512


def _e1_body(reto_ref, xt_ref, att_ref):
    rb = reto_ref[...]
    b = rb[:, :, 192:193]
    m = jnp.max(b, axis=0)
    e = [jnp.exp(b[h] - m) for h in range(N_HASHES)]
    s = e[0] + e[1] + e[2] + e[3]
    acc = xt_ref[...]
    for h in range(N_HASHES):
        acc += (e[h] / s) * rb[h, :, 0:192]
    att_ref[...] = acc


def _run_e1(reto, xt):
    return _pc(
        _e1_body,
        grid=(L // EB,),
        in_specs=[
            pl.BlockSpec((N_HASHES, EB, 256), lambda i: (0, i, 0)),
            pl.BlockSpec((EB, 192), lambda i: (i, 0)),
        ],
        out_specs=pl.BlockSpec((EB, 192), lambda i: (i, 0)),
        out_shape=jax.ShapeDtypeStruct((L, 192), jnp.float32),
    )(reto, xt)


def _e2_body(ap_ref, wc_ref, bc_ref, out_ref):
    i = pl.program_id(0)
    acc = _shifted_slice(ap_ref, 64 + 2500, i) + bc_ref[...]
    for k in range(27):
        dt, r = divmod(k, 9)
        dh, dw = divmod(r, 3)
        off = (dt - 1) * 2500 + (dh - 1) * 50 + (dw - 1)
        sl = jnp.maximum(_shifted_slice(ap_ref, 64 + 2500 + off, i), 0.0)
        acc += jnp.dot(sl, wc_ref[k], preferred_element_type=jnp.float32)
    out_ref[...] = acc


def _run_e2(ap, wc, bc):
    return _pc(
        _e2_body,
        grid=(RNG // RB,),
        in_specs=[
            pl.BlockSpec((PADF, 192), lambda i: (0, 0)),
            pl.BlockSpec((27, 192, 192), lambda i: (0, 0, 0)),
            pl.BlockSpec((1, 192), lambda i: (0, 0)),
        ],
        out_specs=pl.BlockSpec((RB, 192), lambda i: (i, 0)),
        out_shape=jax.ShapeDtypeStruct((RNG, 192), jnp.float32),
    )(ap, wc, bc)


def _flat_pad(chw):
    p5 = jnp.pad(chw, ((0, 0), (1, 1), (1, 1), (1, 1)))
    return jnp.pad(p5.reshape(192, 20000).T, ((64, 64), (0, 0)))


def _interior(rng_rows):
    c = rng_rows.shape[-1]
    return rng_rows.reshape(6, 50, 50, c)[:, 1:49, 1:49, :].reshape(L, c)


def kernel(x, rotations, W_match, b_match, W_assembly, b_assembly,
           W_conv, b_conv):
    xt = x[0].reshape(192, L).T
    xp = _flat_pad(x[0])
    wm = W_match.transpose(2, 3, 4, 1, 0).reshape(27, 192, 48)
    wc = W_conv.transpose(2, 3, 4, 1, 0).reshape(27, 192, 192)
    wa = W_assembly[:, :, 0, 0, 0].T
    rot = rotations[0].transpose(1, 0, 2)

    emb_r = _run_a1(xp, wm, b_match.reshape(1, 48),
                    wa, b_assembly.reshape(1, 192))
    emb = _interior(emb_r)

    rank_l, rank_g = _run_a2(emb, rot)
    del rank_l
    rkg4 = rank_g.reshape(N_HASHES, NW, NCK, CK)

    es = _run_b(emb, rkg4)
    ret_s = _run_c(es.reshape(N_HASHES, NB, CHUNK, 256))
    reto = _run_d(ret_s.reshape(N_HASHES * L, 256), rkg4)

    att = _run_e1(reto, xt)
    ap = _flat_pad(att.T.reshape(192, 6, 48, 48))
    final = _interior(_run_e2(ap, wc, b_conv.reshape(1, 192)))
    return final.T.reshape(1, 192, 6, 48, 48)


# speedup vs baseline: 3.1196x; 1.5288x over previous
import functools
import jax
import jax.numpy as jnp
from jax import lax
from jax.experimental import pallas as pl
from jax.experimental.pallas import tpu as pltpu
from jax.experimental.pallas import tpu_sc as plsc

N_HASHES = 4
CHUNK = 144
L = 6 * 48 * 48
NB = L // CHUNK
HB = 96
TS, HS = 2800, 56
PADF = 80 + 8 * TS + 80
RNG = 16800
RB = 1200
GD = 80
_INTERPRET = False


def _pc(*a, **k):
    return pl.pallas_call(*a, interpret=_INTERPRET, **k)


def _conv_taps(ref, w_ref, i, n, relu=False):
    accs = []
    for dw in range(3):
        acc = None
        for dt in range(3):
            for dh in range(3):
                off = (dt - 1) * TS + (dh - 1) * HS
                al = GD + TS + off - 8
                sl = ref[pl.ds(al + i * RB, RB + 16), :]
                if relu:
                    sl = jnp.maximum(sl, 0.0)
                k = dt * 9 + dh * 3 + dw
                d = jnp.dot(sl, w_ref[k], preferred_element_type=jnp.float32)
                acc = d if acc is None else acc + d
        accs.append(acc)
    return (accs[0][7:7 + RB] + accs[1][8:8 + RB] + accs[2][9:9 + RB])


def _a1_body(xp_ref, wm_ref, bm_ref, wa_ref, ba_ref, emb_ref):
    i = pl.program_id(0)
    xe = _conv_taps(xp_ref, wm_ref, i, 48) + bm_ref[...]
    c = xp_ref[pl.ds(GD + TS + i * RB, RB), :]
    ye = jnp.dot(c, wa_ref[...],
                 preferred_element_type=jnp.float32) + ba_ref[...]
    emb_ref[...] = jnp.concatenate(
        [xe, ye, jnp.zeros((RB, 16), jnp.float32)], axis=-1)


def _run_a1(xp, wm, bm, wa, ba):
    return _pc(
        _a1_body,
        grid=(RNG // RB,),
        in_specs=[
            pl.BlockSpec((PADF, 192), lambda i: (0, 0)),
            pl.BlockSpec((27, 192, 48), lambda i: (0, 0, 0)),
            pl.BlockSpec((1, 48), lambda i: (0, 0)),
            pl.BlockSpec((192, 192), lambda i: (0, 0)),
            pl.BlockSpec((1, 192), lambda i: (0, 0)),
        ],
        out_specs=pl.BlockSpec((RB, 256), lambda i: (i, 0)),
        out_shape=jax.ShapeDtypeStruct((RNG, 256), jnp.float32),
    )(xp, wm, bm, wa, ba)


SB = 512


def _a2_body(xe_ref, rot_ref, rl_ref, rg_ref):
    h = pl.program_id(0)
    xe = xe_ref[...]
    v = jnp.dot(xe, rot_ref[0], preferred_element_type=jnp.float32)
    vv = jnp.concatenate([v, -v], axis=-1)
    m = jnp.max(vv, axis=-1, keepdims=True)
    lane = lax.broadcasted_iota(jnp.int32, (L, HB), 1)
    code = jnp.min(jnp.where(vv >= m, lane, HB), axis=-1, keepdims=True)
    lane128 = lax.broadcasted_iota(jnp.int32, (L, 128), 1)
    onehot = (code == lane128).astype(jnp.float32)

    r = lax.broadcasted_iota(jnp.int32, (SB, SB), 0)
    c = lax.broadcasted_iota(jnp.int32, (SB, SB), 1)
    ltri = (r >= c).astype(jnp.bfloat16)
    r2 = lax.broadcasted_iota(jnp.int32, (128, 128), 0)
    c2 = lax.broadcasted_iota(jnp.int32, (128, 128), 1)
    sup = (r2 < c2).astype(jnp.float32)

    hist = jnp.sum(onehot, axis=0, keepdims=True)
    less = jnp.dot(hist, sup, preferred_element_type=jnp.float32)
    run = jnp.zeros((1, 128), jnp.float32)
    for b in range(L // SB):
        ob = onehot[b * SB:(b + 1) * SB, :]
        incl = jnp.dot(ltri, ob.astype(jnp.bfloat16),
                       preferred_element_type=jnp.float32)
        val = incl - 1.0 + run + less
        rank = jnp.sum(val * ob, axis=-1, keepdims=True)
        ri = rank.astype(jnp.int32)
        rl_ref[0, pl.ds(b * SB, SB), :] = ri
        rg_ref[0, pl.ds(b * SB, SB), :] = ri + h * L
        run = run + incl[SB - 1:SB, :]


def _run_a2(xe, rot):
    return _pc(
        _a2_body,
        grid=(N_HASHES,),
        in_specs=[
            pl.BlockSpec((L, 48), lambda h: (0, 0)),
            pl.BlockSpec((1, 48, 48), lambda h: (h, 0, 0)),
        ],
        out_specs=[
            pl.BlockSpec((1, L, 1), lambda h: (h, 0, 0)),
            pl.BlockSpec((1, L, 1), lambda h: (h, 0, 0)),
        ],
        out_shape=[
            jax.ShapeDtypeStruct((N_HASHES, L, 1), jnp.int32),
            jax.ShapeDtypeStruct((N_HASHES, L, 1), jnp.int32),
        ],
    )(xe, rot)


KT = 8


def _c_body(e0_ref, em_ref, ep_ref, ret_ref):
    lane = lax.broadcasted_iota(jnp.int32, (CHUNK, 256), 1)
    for jj in range(KT):
        cur = e0_ref[0, jj]
        prv = e0_ref[0, jj - 1] if jj > 0 else em_ref[0, 0]
        nxt = e0_ref[0, jj + 1] if jj < KT - 1 else ep_ref[0, 0]
        blks = [cur, prv, nxt]
        xn = []
        for t in blks:
            xx = t[:, 0:48]
            nrm = jnp.sqrt(jnp.sum(xx * xx, axis=-1, keepdims=True))
            xn.append(xx / jnp.clip(nrm, 5e-5, None))
        xmatch = jnp.concatenate(xn, axis=0)
        xa = cur[:, 0:48]
        raw = lax.dot_general(xa, xmatch, (((1,), (1,)), ((), ())),
                              preferred_element_type=jnp.float32)
        m = jnp.max(raw, axis=-1, keepdims=True)
        e = jnp.exp(raw - m)
        s = jnp.sum(e, axis=-1, keepdims=True)
        bs = jnp.log(s) + m
        score = e / s
        ym = jnp.concatenate(blks, axis=0)
        retf = jnp.dot(score, ym, preferred_element_type=jnp.float32)
        ret_ref[0, jj] = jnp.where(lane == 240, bs, retf)


def _run_c(es4):
    return _pc(
        _c_body,
        grid=(N_HASHES, NB // KT),
        in_specs=[
            pl.BlockSpec((1, KT, CHUNK, 256), lambda h, j: (h, j, 0, 0)),
            pl.BlockSpec((1, 1, CHUNK, 256),
                         lambda h, j: (h, (j * KT + NB - 1) % NB, 0, 0)),
            pl.BlockSpec((1, 1, CHUNK, 256),
                         lambda h, j: (h, (j * KT + KT) % NB, 0, 0)),
        ],
        out_specs=pl.BlockSpec((1, KT, CHUNK, 256), lambda h, j: (h, j, 0, 0)),
        out_shape=jax.ShapeDtypeStruct((N_HASHES, NB, CHUNK, 256),
                                       jnp.float32),
    )(es4, es4, es4)


NW = 32
PB = L // NW
CK = 72
NCK = PB // CK


def _sc_wid():
    return lax.axis_index("s") * 2 + lax.axis_index("c")


def _b_body(emb_hbm, rk_hbm, pi_hbm, es_hbm, idx_v, sidx_v, er_v, sem):
    wid = _sc_wid()
    pltpu.sync_copy(pi_hbm.at[wid], sidx_v)
    gps = []
    for c in range(NCK):
        gps.append(pltpu.async_copy(
            emb_hbm.at[sidx_v.at[c]], er_v.at[pl.ds(c * CK, CK)], sem))
    for gp in gps:
        gp.wait()
    for h in range(N_HASHES):
        pltpu.sync_copy(rk_hbm.at[h, wid], idx_v)
        cps = []
        for c in range(NCK):
            cps.append(pltpu.async_copy(
                er_v.at[pl.ds(c * CK, CK)], es_hbm.at[idx_v.at[c]], sem))
        for cp in cps:
            cp.wait()


def _run_b(emb_r, rk4, pidx):
    mesh = plsc.VectorSubcoreMesh(core_axis_name="c", subcore_axis_name="s")
    f = pl.kernel(
        _b_body, mesh=mesh,
        out_type=jax.ShapeDtypeStruct((N_HASHES * L, 256), jnp.float32),
        scratch_types=[pltpu.VMEM((NCK, CK), jnp.int32),
                       pltpu.VMEM((NCK, CK), jnp.int32),
                       pltpu.VMEM((PB, 256), jnp.float32),
                       pltpu.SemaphoreType.DMA],
    )
    return f(emb_r, rk4, pidx)


def _d_body(ret_hbm, rkg_hbm, reto_hbm, idx_v, rows_v, sem):
    wid = _sc_wid()
    base = wid * PB
    for h in range(N_HASHES):
        pltpu.sync_copy(rkg_hbm.at[h, wid], idx_v)
        cps = []
        for c in range(NCK):
            cps.append(pltpu.async_copy(
                ret_hbm.at[idx_v.at[c]], rows_v.at[pl.ds(c * CK, CK)], sem))
        for cp in cps:
            cp.wait()
        pltpu.sync_copy(rows_v, reto_hbm.at[h, pl.ds(base, PB)])


def _run_d(ret_f, rkg4):
    mesh = plsc.VectorSubcoreMesh(core_axis_name="c", subcore_axis_name="s")
    f = pl.kernel(
        _d_body, mesh=mesh,
        out_type=jax.ShapeDtypeStruct((N_HASHES, L, 256), jnp.float32),
        scratch_types=[pltpu.VMEM((NCK, CK), jnp.int32),
                       pltpu.VMEM((PB, 256), jnp.float32),
                       pltpu.SemaphoreType.DMA],
    )
    return f(ret_f, rkg4)


EB = 512


def _e1_body(reto_ref, xt_ref, att_ref, ratt_ref):
    rb = reto_ref[...]
    b = rb[:, :, 240:241]
    m = jnp.max(b, axis=0)
    e = [jnp.exp(b[h] - m) for h in range(N_HASHES)]
    s = e[0] + e[1] + e[2] + e[3]
    acc = (e[0] / s) * rb[0]
    for h in range(1, N_HASHES):
        acc += (e[h] / s) * rb[h]
    att = acc[:, 48:240] + xt_ref[...]
    att_ref[...] = att
    ratt_ref[...] = jnp.maximum(att, 0.0)


def _run_e1(reto, xt):
    return _pc(
        _e1_body,
        grid=(L // EB,),
        in_specs=[
            pl.BlockSpec((N_HASHES, EB, 256), lambda i: (0, i, 0)),
            pl.BlockSpec((EB, 192), lambda i: (i, 0)),
        ],
        out_specs=[pl.BlockSpec((EB, 192), lambda i: (i, 0)),
                   pl.BlockSpec((EB, 192), lambda i: (i, 0))],
        out_shape=[jax.ShapeDtypeStruct((L, 192), jnp.float32),
                   jax.ShapeDtypeStruct((L, 192), jnp.float32)],
    )(reto, xt)


def _e2_body(rp_ref, wc_ref, bc_ref, out_ref):
    i = pl.program_id(0)
    out_ref[...] = _conv_taps(rp_ref, wc_ref, i, 192) + bc_ref[...]


def _run_e2(rp, wc, bc):
    return _pc(
        _e2_body,
        grid=(RNG // RB,),
        in_specs=[
            pl.BlockSpec((PADF, 192), lambda i: (0, 0)),
            pl.BlockSpec((27, 192, 192), lambda i: (0, 0, 0)),
            pl.BlockSpec((1, 192), lambda i: (0, 0)),
        ],
        out_specs=pl.BlockSpec((RB, 192), lambda i: (i, 0)),
        out_shape=jax.ShapeDtypeStruct((RNG, 192), jnp.float32),
    )(rp, wc, bc)


def _f_body(att_ref, cv_ref, out_ref):
    out_ref[...] = att_ref[...] + cv_ref[...]


def _run_f(att, cv):
    return _pc(
        _f_body,
        grid=(L // 1728,),
        in_specs=[pl.BlockSpec((1728, 192), lambda i: (i, 0)),
                  pl.BlockSpec((1728, 192), lambda i: (i, 0))],
        out_specs=pl.BlockSpec((1728, 192), lambda i: (i, 0)),
        out_shape=jax.ShapeDtypeStruct((L, 192), jnp.float32),
    )(att, cv)


def _flat_pad(chw):
    p5 = jnp.pad(chw, ((0, 0), (1, 1), (1, 1), (1, 7)))
    return jnp.pad(p5.reshape(192, 8 * TS).T, ((GD, GD), (0, 0)))


def _interior(rng_rows):
    c = rng_rows.shape[-1]
    return rng_rows.reshape(6, 50, HS, c)[:, 1:49, 1:49, :].reshape(L, c)


def _pad_indices():
    ii = jnp.arange(L, dtype=jnp.int32)
    row = ((ii // 2304) * TS + ((ii % 2304) // 48 + 1) * HS + (ii % 48) + 1)
    return row.reshape(NW, NCK, CK)


def kernel(x, rotations, W_match, b_match, W_assembly, b_assembly,
           W_conv, b_conv):
    xt = x[0].reshape(192, L).T
    xp = _flat_pad(x[0])
    wm = W_match.transpose(2, 3, 4, 1, 0).reshape(27, 192, 48)
    wc = W_conv.transpose(2, 3, 4, 1, 0).reshape(27, 192, 192)
    wa = W_assembly[:, :, 0, 0, 0].T
    rot = rotations[0].transpose(1, 0, 2)

    emb_r = _run_a1(xp, wm, b_match.reshape(1, 48),
                    wa, b_assembly.reshape(1, 192))

    rank_l, rank_g = _run_a2(_interior(emb_r[:, 0:48]), rot)
    del rank_l
    rkg4 = rank_g.reshape(N_HASHES, NW, NCK, CK)

    es = _run_b(emb_r, rkg4, _pad_indices())
    ret_s = _run_c(es.reshape(N_HASHES, NB, CHUNK, 256))
    reto = _run_d(ret_s.reshape(N_HASHES * L, 256), rkg4)

    att, ratt = _run_e1(reto, xt)
    rp = _flat_pad(ratt.T.reshape(192, 6, 48, 48))
    cv = _interior(_run_e2(rp, wc, b_conv.reshape(1, 192)))
    final = _run_f(att, cv)
    return final.T.reshape(1, 192, 6, 48, 48)


# speedup vs baseline: 3.3411x; 1.0710x over previous
import functools
import jax
import jax.numpy as jnp
from jax import lax
from jax.experimental import pallas as pl
from jax.experimental.pallas import tpu as pltpu
from jax.experimental.pallas import tpu_sc as plsc

N_HASHES = 4
CHUNK = 144
L = 6 * 48 * 48
NB = L // CHUNK
HB = 96
TS, HS = 2800, 56
PADF = 80 + 8 * TS + 80
RNG = 16800
RB = 1200
GD = 80
_INTERPRET = False


def _pc(*a, **k):
    return pl.pallas_call(*a, interpret=_INTERPRET, **k)


def _conv_taps(ref, w_ref, i, n, relu=False):
    accs = []
    for dw in range(3):
        acc = None
        for dt in range(3):
            for dh in range(3):
                off = (dt - 1) * TS + (dh - 1) * HS
                al = GD + TS + off - 8
                sl = ref[pl.ds(al + i * RB, RB + 16), :]
                if relu:
                    sl = jnp.maximum(sl, 0.0)
                k = dt * 9 + dh * 3 + dw
                d = jnp.dot(sl, w_ref[k], preferred_element_type=jnp.float32)
                acc = d if acc is None else acc + d
        accs.append(acc)
    return (accs[0][7:7 + RB] + accs[1][8:8 + RB] + accs[2][9:9 + RB])


def _a1_body(xp_ref, wm_ref, bm_ref, wa_ref, ba_ref, emb_ref):
    i = pl.program_id(0)
    xe = _conv_taps(xp_ref, wm_ref, i, 48) + bm_ref[...]
    c = xp_ref[pl.ds(GD + TS + i * RB, RB), :]
    ye = jnp.dot(c, wa_ref[...],
                 preferred_element_type=jnp.float32) + ba_ref[...]
    emb_ref[...] = jnp.concatenate(
        [xe, ye, jnp.zeros((RB, 16), jnp.float32)], axis=-1)


def _run_a1(xp, wm, bm, wa, ba):
    return _pc(
        _a1_body,
        grid=(RNG // RB,),
        in_specs=[
            pl.BlockSpec((PADF, 192), lambda i: (0, 0)),
            pl.BlockSpec((27, 192, 48), lambda i: (0, 0, 0)),
            pl.BlockSpec((1, 48), lambda i: (0, 0)),
            pl.BlockSpec((192, 192), lambda i: (0, 0)),
            pl.BlockSpec((1, 192), lambda i: (0, 0)),
        ],
        out_specs=pl.BlockSpec((RB, 256), lambda i: (i, 0)),
        out_shape=jax.ShapeDtypeStruct((RNG, 256), jnp.float32),
    )(xp, wm, bm, wa, ba)


SB = 512


def _a2_body(xe_ref, rot_ref, rg_ref):
    h = pl.program_id(0)
    xe = xe_ref[...]
    v = jnp.dot(xe, rot_ref[0], preferred_element_type=jnp.float32)
    vv = jnp.concatenate([v, -v], axis=-1)
    m = jnp.max(vv, axis=-1, keepdims=True)
    lane = lax.broadcasted_iota(jnp.int32, (L, HB), 1)
    code = jnp.min(jnp.where(vv >= m, lane, HB), axis=-1, keepdims=True)
    lane128 = lax.broadcasted_iota(jnp.int32, (L, 128), 1)
    eq = code == lane128
    onehot = eq.astype(jnp.bfloat16)

    r = lax.broadcasted_iota(jnp.int32, (SB, SB), 0)
    c = lax.broadcasted_iota(jnp.int32, (SB, SB), 1)
    ltri = (r >= c).astype(jnp.bfloat16)
    r2 = lax.broadcasted_iota(jnp.int32, (128, 128), 0)
    c2 = lax.broadcasted_iota(jnp.int32, (128, 128), 1)
    sup = (r2 < c2).astype(jnp.float32)

    ones_row = jnp.ones((8, L), jnp.bfloat16)
    hist = jnp.dot(ones_row, onehot,
                   preferred_element_type=jnp.float32)[0:1]
    less = jnp.dot(hist, sup, preferred_element_type=jnp.float32)
    run = jnp.zeros((1, 128), jnp.float32)
    for b in range(L // SB):
        ob = onehot[b * SB:(b + 1) * SB, :]
        incl = jnp.dot(ltri, ob, preferred_element_type=jnp.float32)
        val = incl - 1.0 + run + less + jnp.float32(h * L)
        rank = jnp.sum(jnp.where(eq[b * SB:(b + 1) * SB, :], val, 0.0),
                       axis=-1, keepdims=True)
        rg_ref[0, pl.ds(b * SB, SB), :] = rank.astype(jnp.int32)
        run = run + incl[SB - 1:SB, :]


def _run_a2(xe, rot):
    return _pc(
        _a2_body,
        grid=(N_HASHES,),
        in_specs=[
            pl.BlockSpec((L, 48), lambda h: (0, 0)),
            pl.BlockSpec((1, 48, 48), lambda h: (h, 0, 0)),
        ],
        out_specs=pl.BlockSpec((1, L, 1), lambda h: (h, 0, 0)),
        out_shape=jax.ShapeDtypeStruct((N_HASHES, L, 1), jnp.int32),
    )(xe, rot)


KT = 8


def _c_body(e0_ref, em_ref, ep_ref, ret_ref):
    lane = lax.broadcasted_iota(jnp.int32, (CHUNK, 256), 1)
    for jj in range(KT):
        cur = e0_ref[0, jj]
        prv = e0_ref[0, jj - 1] if jj > 0 else em_ref[0, 0]
        nxt = e0_ref[0, jj + 1] if jj < KT - 1 else ep_ref[0, 0]
        blks = [cur, prv, nxt]
        xa = cur[:, 0:48]
        raws = []
        for t in blks:
            xx = t[:, 0:48]
            ss = jnp.sum(xx * xx, axis=-1, keepdims=True)
            xn = xx * lax.rsqrt(jnp.maximum(ss, 2.5e-9))
            raws.append(lax.dot_general(
                xa, xn, (((1,), (1,)), ((), ())),
                preferred_element_type=jnp.float32))
        m = jnp.maximum(jnp.maximum(
            jnp.max(raws[0], axis=-1, keepdims=True),
            jnp.max(raws[1], axis=-1, keepdims=True)),
            jnp.max(raws[2], axis=-1, keepdims=True))
        es_ = [jnp.exp(r - m) for r in raws]
        s = (jnp.sum(es_[0], axis=-1, keepdims=True)
             + jnp.sum(es_[1], axis=-1, keepdims=True)
             + jnp.sum(es_[2], axis=-1, keepdims=True))
        bs = jnp.log(s) + m
        acc = jnp.dot(es_[0], blks[0], preferred_element_type=jnp.float32)
        acc += jnp.dot(es_[1], blks[1], preferred_element_type=jnp.float32)
        acc += jnp.dot(es_[2], blks[2], preferred_element_type=jnp.float32)
        retf = acc * (1.0 / s)
        ret_ref[0, jj] = jnp.where(lane == 240, bs, retf)


def _run_c(es4):
    return _pc(
        _c_body,
        grid=(N_HASHES, NB // KT),
        in_specs=[
            pl.BlockSpec((1, KT, CHUNK, 256), lambda h, j: (h, j, 0, 0)),
            pl.BlockSpec((1, 1, CHUNK, 256),
                         lambda h, j: (h, (j * KT + NB - 1) % NB, 0, 0)),
            pl.BlockSpec((1, 1, CHUNK, 256),
                         lambda h, j: (h, (j * KT + KT) % NB, 0, 0)),
        ],
        out_specs=pl.BlockSpec((1, KT, CHUNK, 256), lambda h, j: (h, j, 0, 0)),
        out_shape=jax.ShapeDtypeStruct((N_HASHES, NB, CHUNK, 256),
                                       jnp.float32),
    )(es4, es4, es4)


NW = 32
PB = L // NW
CK = 72
NCK = PB // CK


def _sc_wid():
    return lax.axis_index("s") * 2 + lax.axis_index("c")


def _b_body(emb_hbm, rk_hbm, pi_hbm, es_hbm, idx_v, sidx_v, er_v, sem):
    wid = _sc_wid()
    pltpu.sync_copy(pi_hbm.at[wid], sidx_v)
    gps = []
    for c in range(NCK):
        gps.append(pltpu.async_copy(
            emb_hbm.at[sidx_v.at[c]], er_v.at[pl.ds(c * CK, CK)], sem))
    for gp in gps:
        gp.wait()
    for h in range(N_HASHES):
        pltpu.sync_copy(rk_hbm.at[h, wid], idx_v)
        cps = []
        for c in range(NCK):
            cps.append(pltpu.async_copy(
                er_v.at[pl.ds(c * CK, CK)], es_hbm.at[idx_v.at[c]], sem))
        for cp in cps:
            cp.wait()


def _run_b(emb_r, rk4, pidx):
    mesh = plsc.VectorSubcoreMesh(core_axis_name="c", subcore_axis_name="s")
    f = pl.kernel(
        _b_body, mesh=mesh,
        out_type=jax.ShapeDtypeStruct((N_HASHES * L, 256), jnp.float32),
        scratch_types=[pltpu.VMEM((NCK, CK), jnp.int32),
                       pltpu.VMEM((NCK, CK), jnp.int32),
                       pltpu.VMEM((PB, 256), jnp.float32),
                       pltpu.SemaphoreType.DMA],
    )
    return f(emb_r, rk4, pidx)


def _d_body(ret_hbm, rkg_hbm, reto_hbm, idx_v, rows_v, sem):
    wid = _sc_wid()
    base = wid * PB
    for h in range(N_HASHES):
        pltpu.sync_copy(rkg_hbm.at[h, wid], idx_v)
        cps = []
        for c in range(NCK):
            cps.append(pltpu.async_copy(
                ret_hbm.at[idx_v.at[c]], rows_v.at[pl.ds(c * CK, CK)], sem))
        for cp in cps:
            cp.wait()
        pltpu.sync_copy(rows_v, reto_hbm.at[h, pl.ds(base, PB)])


def _run_d(ret_f, rkg4):
    mesh = plsc.VectorSubcoreMesh(core_axis_name="c", subcore_axis_name="s")
    f = pl.kernel(
        _d_body, mesh=mesh,
        out_type=jax.ShapeDtypeStruct((N_HASHES, L, 256), jnp.float32),
        scratch_types=[pltpu.VMEM((NCK, CK), jnp.int32),
                       pltpu.VMEM((PB, 256), jnp.float32),
                       pltpu.SemaphoreType.DMA],
    )
    return f(ret_f, rkg4)


EB = 512


def _e1_body(reto_ref, xc_ref, att_ref, ratt_ref):
    rb = reto_ref[...]
    b = rb[:, :, 240:241]
    m = jnp.max(b, axis=0)
    e = [jnp.exp(b[h] - m) for h in range(N_HASHES)]
    s = e[0] + e[1] + e[2] + e[3]
    acc = (e[0] / s) * rb[0]
    for h in range(1, N_HASHES):
        acc += (e[h] / s) * rb[h]
    att = acc[:, 48:240] + jnp.transpose(xc_ref[...])
    att_ref[...] = att
    ratt_ref[...] = jnp.maximum(att, 0.0).astype(jnp.bfloat16)


def _run_e1(reto, xc):
    return _pc(
        _e1_body,
        grid=(L // EB,),
        in_specs=[
            pl.BlockSpec((N_HASHES, EB, 256), lambda i: (0, i, 0)),
            pl.BlockSpec((192, EB), lambda i: (0, i)),
        ],
        out_specs=[pl.BlockSpec((EB, 192), lambda i: (i, 0)),
                   pl.BlockSpec((EB, 192), lambda i: (i, 0))],
        out_shape=[jax.ShapeDtypeStruct((L, 192), jnp.float32),
                   jax.ShapeDtypeStruct((L, 192), jnp.bfloat16)],
    )(reto, xc)


def _e2_body(rp_ref, wc_ref, bc_ref, out_ref):
    i = pl.program_id(0)
    out_ref[...] = _conv_taps(rp_ref, wc_ref, i, 192) + bc_ref[...]


def _run_e2(rp, wc, bc):
    return _pc(
        _e2_body,
        grid=(RNG // RB,),
        in_specs=[
            pl.BlockSpec((PADF, 192), lambda i: (0, 0)),
            pl.BlockSpec((27, 192, 192), lambda i: (0, 0, 0)),
            pl.BlockSpec((1, 192), lambda i: (0, 0)),
        ],
        out_specs=pl.BlockSpec((RB, 192), lambda i: (i, 0)),
        out_shape=jax.ShapeDtypeStruct((RNG, 192), jnp.float32),
    )(rp, wc, bc)


def _f_body(att_ref, cv_ref, out_ref):
    out_ref[...] = jnp.transpose(att_ref[...] + cv_ref[...])


def _run_f(att, cv):
    return _pc(
        _f_body,
        grid=(L // 2304,),
        in_specs=[pl.BlockSpec((2304, 192), lambda i: (i, 0)),
                  pl.BlockSpec((2304, 192), lambda i: (i, 0))],
        out_specs=pl.BlockSpec((192, 2304), lambda i: (0, i)),
        out_shape=jax.ShapeDtypeStruct((192, L), jnp.float32),
    )(att, cv)


def _flat_pad(chw):
    p5 = jnp.pad(chw, ((0, 0), (1, 1), (1, 1), (1, 7)))
    return jnp.pad(p5.reshape(192, 8 * TS).T, ((GD, GD), (0, 0)))


def _interior(rng_rows):
    c = rng_rows.shape[-1]
    return rng_rows.reshape(6, 50, HS, c)[:, 1:49, 1:49, :].reshape(L, c)


def _pad_indices():
    ii = jnp.arange(L, dtype=jnp.int32)
    row = ((ii // 2304) * TS + ((ii % 2304) // 48 + 1) * HS + (ii % 48) + 1)
    return row.reshape(NW, NCK, CK)


def kernel(x, rotations, W_match, b_match, W_assembly, b_assembly,
           W_conv, b_conv):
    xc = x[0].reshape(192, L)
    xp = _flat_pad(x[0])
    wm = W_match.transpose(2, 3, 4, 1, 0).reshape(27, 192, 48)
    wc = W_conv.transpose(2, 3, 4, 1, 0).reshape(27, 192, 192)
    wa = W_assembly[:, :, 0, 0, 0].T
    rot = rotations[0].transpose(1, 0, 2)

    emb_r = _run_a1(xp, wm, b_match.reshape(1, 48),
                    wa, b_assembly.reshape(1, 192))

    rank_g = _run_a2(_interior(emb_r[:, 0:48]), rot)
    rkg4 = rank_g.reshape(N_HASHES, NW, NCK, CK)

    es = _run_b(emb_r, rkg4, _pad_indices())
    ret_s = _run_c(es.reshape(N_HASHES, NB, CHUNK, 256))
    reto = _run_d(ret_s.reshape(N_HASHES * L, 256), rkg4)

    att, ratt = _run_e1(reto, xc)
    rp = _flat_pad(ratt.T.reshape(192, 6, 48, 48))
    cv = _interior(_run_e2(rp, wc.astype(jnp.bfloat16),
                           b_conv.reshape(1, 192)))
    final = _run_f(att, cv)
    return final.reshape(1, 192, 6, 48, 48)


# speedup vs baseline: 3.5818x; 1.0720x over previous
import functools
import jax
import jax.numpy as jnp
from jax import lax
from jax.experimental import pallas as pl
from jax.experimental.pallas import tpu as pltpu
from jax.experimental.pallas import tpu_sc as plsc

N_HASHES = 4
CHUNK = 144
L = 6 * 48 * 48
NB = L // CHUNK
HB = 96
TS, HS = 2800, 56
PADF = 80 + 8 * TS + 80
RNG = 16800
RB = 1200
GD = 80
_INTERPRET = False


def _pc(*a, **k):
    return pl.pallas_call(*a, interpret=_INTERPRET, **k)


def _conv_taps(ref, w_ref, i, n, relu=False):
    accs = []
    for dw in range(3):
        acc = None
        for dt in range(3):
            for dh in range(3):
                off = (dt - 1) * TS + (dh - 1) * HS
                al = GD + TS + off - 8
                sl = ref[pl.ds(al + i * RB, RB + 16), :]
                if relu:
                    sl = jnp.maximum(sl, 0.0)
                k = dt * 9 + dh * 3 + dw
                d = jnp.dot(sl, w_ref[k], preferred_element_type=jnp.float32)
                acc = d if acc is None else acc + d
        accs.append(acc)
    return (accs[0][7:7 + RB] + accs[1][8:8 + RB] + accs[2][9:9 + RB])


def _a1_body(xp_ref, wm_ref, bm_ref, wa_ref, ba_ref, emb_ref):
    i = pl.program_id(0)
    acc = None
    for dt in range(3):
        for dh in range(3):
            off = (dt - 1) * TS + (dh - 1) * HS
            al = GD + TS + off - 8
            sl = xp_ref[pl.ds(al + i * RB, RB + 16), :]
            d = jnp.dot(sl, wm_ref[dt * 3 + dh],
                        preferred_element_type=jnp.float32)
            acc = d if acc is None else acc + d
    xe = (acc[7:7 + RB, 0:48] + acc[8:8 + RB, 48:96]
          + acc[9:9 + RB, 96:144]) + bm_ref[...]
    c = xp_ref[pl.ds(GD + TS + i * RB, RB), :]
    ye = jnp.dot(c, wa_ref[...],
                 preferred_element_type=jnp.float32) + ba_ref[...]
    emb_ref[...] = jnp.concatenate(
        [xe, ye, jnp.zeros((RB, 16), jnp.float32)], axis=-1)


def _run_a1(xp, wm9, bm, wa, ba):
    return _pc(
        _a1_body,
        grid=(RNG // RB,),
        in_specs=[
            pl.BlockSpec((PADF, 192), lambda i: (0, 0)),
            pl.BlockSpec((9, 192, 144), lambda i: (0, 0, 0)),
            pl.BlockSpec((1, 48), lambda i: (0, 0)),
            pl.BlockSpec((192, 192), lambda i: (0, 0)),
            pl.BlockSpec((1, 192), lambda i: (0, 0)),
        ],
        out_specs=pl.BlockSpec((RB, 256), lambda i: (i, 0)),
        out_shape=jax.ShapeDtypeStruct((RNG, 256), jnp.float32),
    )(xp, wm9, bm, wa, ba)


SB = 512


def _a2_body(xe_ref, rot_ref, rg_ref):
    h = pl.program_id(0)
    xe = xe_ref[...]
    v = jnp.dot(xe, rot_ref[0], preferred_element_type=jnp.float32)
    vv = jnp.concatenate([v, -v], axis=-1)
    m = jnp.max(vv, axis=-1, keepdims=True)
    lane = lax.broadcasted_iota(jnp.int32, (L, HB), 1)
    code = jnp.min(jnp.where(vv >= m, lane, HB), axis=-1, keepdims=True)
    lane128 = lax.broadcasted_iota(jnp.int32, (L, 128), 1)
    eq = code == lane128
    onehot = eq.astype(jnp.bfloat16)

    r = lax.broadcasted_iota(jnp.int32, (SB, SB), 0)
    c = lax.broadcasted_iota(jnp.int32, (SB, SB), 1)
    ltri = (r >= c).astype(jnp.bfloat16)
    r2 = lax.broadcasted_iota(jnp.int32, (128, 128), 0)
    c2 = lax.broadcasted_iota(jnp.int32, (128, 128), 1)
    sup = (r2 < c2).astype(jnp.float32)

    ones_row = jnp.ones((8, L), jnp.bfloat16)
    hist = jnp.dot(ones_row, onehot,
                   preferred_element_type=jnp.float32)[0:1]
    less = jnp.dot(hist, sup, preferred_element_type=jnp.float32)
    run = jnp.zeros((1, 128), jnp.float32)
    for b in range(L // SB):
        ob = onehot[b * SB:(b + 1) * SB, :]
        incl = jnp.dot(ltri, ob, preferred_element_type=jnp.float32)
        val = incl - 1.0 + run + less + jnp.float32(h * L)
        rank = jnp.sum(jnp.where(eq[b * SB:(b + 1) * SB, :], val, 0.0),
                       axis=-1, keepdims=True)
        rg_ref[0, pl.ds(b * SB, SB), :] = rank.astype(jnp.int32)
        run = run + incl[SB - 1:SB, :]


def _run_a2(xe, rot):
    return _pc(
        _a2_body,
        grid=(N_HASHES,),
        in_specs=[
            pl.BlockSpec((L, 48), lambda h: (0, 0)),
            pl.BlockSpec((1, 48, 48), lambda h: (h, 0, 0)),
        ],
        out_specs=pl.BlockSpec((1, L, 1), lambda h: (h, 0, 0)),
        out_shape=jax.ShapeDtypeStruct((N_HASHES, L, 1), jnp.int32),
    )(xe, rot)


KT = 8


def _c_body(e0_ref, em_ref, ep_ref, ret_ref):
    lane = lax.broadcasted_iota(jnp.int32, (CHUNK, 256), 1)
    for jj in range(KT):
        cur = e0_ref[0, jj]
        prv = e0_ref[0, jj - 1] if jj > 0 else em_ref[0, 0]
        nxt = e0_ref[0, jj + 1] if jj < KT - 1 else ep_ref[0, 0]
        blks = [cur, prv, nxt]
        xa = cur[:, 0:48]
        raws = []
        for t in blks:
            xx = t[:, 0:48]
            ss = jnp.sum(xx * xx, axis=-1, keepdims=True)
            xn = xx * lax.rsqrt(jnp.maximum(ss, 2.5e-9))
            raws.append(lax.dot_general(
                xa, xn, (((1,), (1,)), ((), ())),
                preferred_element_type=jnp.float32))
        m = jnp.maximum(jnp.maximum(
            jnp.max(raws[0], axis=-1, keepdims=True),
            jnp.max(raws[1], axis=-1, keepdims=True)),
            jnp.max(raws[2], axis=-1, keepdims=True))
        es_ = [jnp.exp(r - m) for r in raws]
        s = (jnp.sum(es_[0], axis=-1, keepdims=True)
             + jnp.sum(es_[1], axis=-1, keepdims=True)
             + jnp.sum(es_[2], axis=-1, keepdims=True))
        bs = jnp.log(s) + m
        acc = jnp.dot(es_[0], blks[0], preferred_element_type=jnp.float32)
        acc += jnp.dot(es_[1], blks[1], preferred_element_type=jnp.float32)
        acc += jnp.dot(es_[2], blks[2], preferred_element_type=jnp.float32)
        retf = acc * (1.0 / s)
        ret_ref[0, jj] = jnp.where(lane == 240, bs, retf)


def _run_c(es4):
    return _pc(
        _c_body,
        grid=(N_HASHES, NB // KT),
        in_specs=[
            pl.BlockSpec((1, KT, CHUNK, 256), lambda h, j: (h, j, 0, 0)),
            pl.BlockSpec((1, 1, CHUNK, 256),
                         lambda h, j: (h, (j * KT + NB - 1) % NB, 0, 0)),
            pl.BlockSpec((1, 1, CHUNK, 256),
                         lambda h, j: (h, (j * KT + KT) % NB, 0, 0)),
        ],
        out_specs=pl.BlockSpec((1, KT, CHUNK, 256), lambda h, j: (h, j, 0, 0)),
        out_shape=jax.ShapeDtypeStruct((N_HASHES, NB, CHUNK, 256),
                                       jnp.float32),
    )(es4, es4, es4)


NW = 32
PB = L // NW
CK = 72
NCK = PB // CK


def _sc_wid():
    return lax.axis_index("s") * 2 + lax.axis_index("c")


def _b_body(emb_hbm, rk_hbm, pi_hbm, es_hbm, idx_v, sidx_v, er_v, sem):
    wid = _sc_wid()
    pltpu.sync_copy(pi_hbm.at[wid], sidx_v)
    gps = []
    for c in range(NCK):
        gps.append(pltpu.async_copy(
            emb_hbm.at[sidx_v.at[c]], er_v.at[pl.ds(c * CK, CK)], sem))
    for gp in gps:
        gp.wait()
    for h in range(N_HASHES):
        pltpu.sync_copy(rk_hbm.at[h, wid], idx_v.at[pl.ds(h * NCK, NCK)])
    cps = []
    for h in range(N_HASHES):
        for c in range(NCK):
            cps.append(pltpu.async_copy(
                er_v.at[pl.ds(c * CK, CK)],
                es_hbm.at[idx_v.at[h * NCK + c]], sem))
    for cp in cps:
        cp.wait()


def _run_b(emb_r, rk4, pidx):
    mesh = plsc.VectorSubcoreMesh(core_axis_name="c", subcore_axis_name="s")
    f = pl.kernel(
        _b_body, mesh=mesh,
        out_type=jax.ShapeDtypeStruct((N_HASHES * L, 256), jnp.float32),
        scratch_types=[pltpu.VMEM((N_HASHES * NCK, CK), jnp.int32),
                       pltpu.VMEM((NCK, CK), jnp.int32),
                       pltpu.VMEM((PB, 256), jnp.float32),
                       pltpu.SemaphoreType.DMA],
    )
    return f(emb_r, rk4, pidx)


def _d_body(ret_hbm, rkg_hbm, reto_hbm, idx_v, rows_v, sem):
    wid = _sc_wid()
    base = wid * PB
    for h in range(N_HASHES):
        pltpu.sync_copy(rkg_hbm.at[h, wid], idx_v)
        cps = []
        for c in range(NCK):
            cps.append(pltpu.async_copy(
                ret_hbm.at[idx_v.at[c]], rows_v.at[pl.ds(c * CK, CK)], sem))
        for cp in cps:
            cp.wait()
        pltpu.sync_copy(rows_v, reto_hbm.at[h, pl.ds(base, PB)])


def _run_d(ret_f, rkg4):
    mesh = plsc.VectorSubcoreMesh(core_axis_name="c", subcore_axis_name="s")
    f = pl.kernel(
        _d_body, mesh=mesh,
        out_type=jax.ShapeDtypeStruct((N_HASHES, L, 256), jnp.float32),
        scratch_types=[pltpu.VMEM((NCK, CK), jnp.int32),
                       pltpu.VMEM((PB, 256), jnp.float32),
                       pltpu.SemaphoreType.DMA],
    )
    return f(ret_f, rkg4)


EB = 512


def _e1_body(reto_ref, xc_ref, att_ref, ratt_ref):
    rb = reto_ref[...]
    b = rb[:, :, 240:241]
    m = jnp.max(b, axis=0)
    e = [jnp.exp(b[h] - m) for h in range(N_HASHES)]
    s = e[0] + e[1] + e[2] + e[3]
    acc = (e[0] / s) * rb[0]
    for h in range(1, N_HASHES):
        acc += (e[h] / s) * rb[h]
    att = acc[:, 48:240] + jnp.transpose(xc_ref[...])
    att_ref[...] = att
    ratt_ref[...] = jnp.maximum(att, 0.0).astype(jnp.bfloat16)


def _run_e1(reto, xc):
    return _pc(
        _e1_body,
        grid=(L // EB,),
        in_specs=[
            pl.BlockSpec((N_HASHES, EB, 256), lambda i: (0, i, 0)),
            pl.BlockSpec((192, EB), lambda i: (0, i)),
        ],
        out_specs=[pl.BlockSpec((EB, 192), lambda i: (i, 0)),
                   pl.BlockSpec((EB, 192), lambda i: (i, 0))],
        out_shape=[jax.ShapeDtypeStruct((L, 192), jnp.float32),
                   jax.ShapeDtypeStruct((L, 192), jnp.bfloat16)],
    )(reto, xc)


def _e2_body(rp_ref, wc_ref, bc_ref, out_ref):
    i = pl.program_id(0)
    out_ref[...] = _conv_taps(rp_ref, wc_ref, i, 192) + bc_ref[...]


def _run_e2(rp, wc, bc):
    return _pc(
        _e2_body,
        grid=(RNG // RB,),
        in_specs=[
            pl.BlockSpec((PADF, 192), lambda i: (0, 0)),
            pl.BlockSpec((27, 192, 192), lambda i: (0, 0, 0)),
            pl.BlockSpec((1, 192), lambda i: (0, 0)),
        ],
        out_specs=pl.BlockSpec((RB, 192), lambda i: (i, 0)),
        out_shape=jax.ShapeDtypeStruct((RNG, 192), jnp.float32),
    )(rp, wc, bc)


def _f_body(att_ref, cv_ref, out_ref):
    out_ref[...] = jnp.transpose(att_ref[...] + cv_ref[...])


def _run_f(att, cv):
    return _pc(
        _f_body,
        grid=(L // 2304,),
        in_specs=[pl.BlockSpec((2304, 192), lambda i: (i, 0)),
                  pl.BlockSpec((2304, 192), lambda i: (i, 0))],
        out_specs=pl.BlockSpec((192, 2304), lambda i: (0, i)),
        out_shape=jax.ShapeDtypeStruct((192, L), jnp.float32),
    )(att, cv)


def _flat_pad(chw):
    p5 = jnp.pad(chw, ((0, 0), (1, 1), (1, 1), (1, 7)))
    return jnp.pad(p5.reshape(192, 8 * TS).T, ((GD, GD), (0, 0)))


def _interior(rng_rows):
    c = rng_rows.shape[-1]
    return rng_rows.reshape(6, 50, HS, c)[:, 1:49, 1:49, :].reshape(L, c)


def _pad_indices():
    ii = jnp.arange(L, dtype=jnp.int32)
    row = ((ii // 2304) * TS + ((ii % 2304) // 48 + 1) * HS + (ii % 48) + 1)
    return row.reshape(NW, NCK, CK)


def kernel(x, rotations, W_match, b_match, W_assembly, b_assembly,
           W_conv, b_conv):
    xc = x[0].reshape(192, L)
    xp = _flat_pad(x[0])
    wm9 = W_match.transpose(2, 3, 4, 1, 0).transpose(0, 1, 3, 2, 4) \
        .reshape(9, 192, 144)
    wc = W_conv.transpose(2, 3, 4, 1, 0).reshape(27, 192, 192)
    wa = W_assembly[:, :, 0, 0, 0].T
    rot = rotations[0].transpose(1, 0, 2)

    emb_r = _run_a1(xp, wm9, b_match.reshape(1, 48),
                    wa, b_assembly.reshape(1, 192))

    rank_g = _run_a2(_interior(emb_r[:, 0:48]), rot)
    rkg4 = rank_g.reshape(N_HASHES, NW, NCK, CK)

    es = _run_b(emb_r, rkg4, _pad_indices())
    ret_s = _run_c(es.reshape(N_HASHES, NB, CHUNK, 256))
    reto = _run_d(ret_s.reshape(N_HASHES * L, 256), rkg4)

    att, ratt = _run_e1(reto, xc)
    rp = _flat_pad(ratt.T.reshape(192, 6, 48, 48))
    cv = _interior(_run_e2(rp, wc.astype(jnp.bfloat16),
                           b_conv.reshape(1, 192)))
    final = _run_f(att, cv)
    return final.reshape(1, 192, 6, 48, 48)
